# baseline (speedup 1.0000x reference)
import functools

import jax
import jax.numpy as jnp
from jax import lax
from jax.experimental import pallas as pl
from jax.experimental.pallas import tpu as pltpu, tpu_sc as plsc

_N = 10000
_NP = 10016
_RSTRIPE = _NP // 16
_E = 320000
_NW = 32
_G = 64

_LAYOUT = {144: (64, 142, 174), 128: (80, 84, 168)}


def _edge_arrays(adjacency, width):
    lb, j0, j1 = _LAYOUT[width]
    ep = 16 * (j0 + j1) * lb
    pad = jnp.full((ep - _E,), _N, jnp.int32)
    srcr = jnp.concatenate([adjacency[0], pad]).reshape(16 * (j0 + j1), lb)
    dstr = jnp.concatenate([adjacency[1], pad]).reshape(16 * (j0 + j1), lb)
    return srcr, dstr


@functools.lru_cache(maxsize=None)
def _make_spmm(width):
    lb, j0, j1 = _LAYOUT[width]
    jmax = max(j0, j1)
    mesh = plsc.VectorSubcoreMesh(core_axis_name="c", subcore_axis_name="s",
                                  num_cores=2, num_subcores=16)

    @functools.partial(
        pl.kernel,
        out_type=jax.ShapeDtypeStruct((2, _NP, width), jnp.float32),
        mesh=mesh,
        scratch_types=[
            pltpu.VMEM_SHARED((_NP, width), jnp.float32),
            pltpu.VMEM((jmax, lb), jnp.int32),
            pltpu.VMEM((jmax, lb), jnp.int32),
            pltpu.VMEM((lb, width), jnp.float32),
            pltpu.VMEM((lb, width), jnp.float32),
            pltpu.SemaphoreType.DMA,
            pltpu.SemaphoreType.DMA,
        ],
        compiler_params=pltpu.CompilerParams(use_tc_tiling_on_sc=False),
    )
    def spmm(table, srcr, dstr, zeros, out, agg_sh, idxs, idxd,
             rows0, rows1, sem0, sem1):
        c = lax.axis_index("c")
        s = lax.axis_index("s")
        pltpu.sync_copy(zeros.at[pl.ds(s * _RSTRIPE, _RSTRIPE)],
                        agg_sh.at[pl.ds(s * _RSTRIPE, _RSTRIPE)])
        jc = jnp.where(c == 0, j0, j1)
        base = jnp.where(c == 0, s * j0, 16 * j0 + s * j1)
        pltpu.sync_copy(srcr.at[pl.ds(base, jmax)], idxs)
        pltpu.sync_copy(dstr.at[pl.ds(base, jmax)], idxd)
        plsc.subcore_barrier()

        pltpu.async_copy(table.at[idxs.at[0]], rows0, sem0)

        def step(p, carry):
            j = p * 2
            pltpu.async_copy(table.at[idxs.at[j + 1]], rows1, sem1)
            pltpu.make_async_copy(table.at[idxs.at[j]], rows0, sem0).wait()
            pltpu.sync_copy(rows0, agg_sh.at[idxd.at[j]], add=True)

            @pl.when(p < jc // 2 - 1)
            def _():
                pltpu.async_copy(table.at[idxs.at[j + 2]], rows0, sem0)

            pltpu.make_async_copy(table.at[idxs.at[j + 1]], rows1, sem1).wait()
            pltpu.sync_copy(rows1, agg_sh.at[idxd.at[j + 1]], add=True)
            return carry

        lax.fori_loop(0, jc // 2, step, 0)
        plsc.subcore_barrier()
        pltpu.sync_copy(agg_sh.at[pl.ds(s * _RSTRIPE, _RSTRIPE)],
                        out.at[c].at[pl.ds(s * _RSTRIPE, _RSTRIPE)])

    return spmm


def _neigh_h(agg2, dsl, x, ws, wn, b):
    p = agg2[0] + agg2[1]
    deg = jnp.sum(dsl[0] + dsl[1], axis=1, keepdims=True)
    neigh = p * (1.0 / jnp.maximum(deg, 1.0))
    h = x @ ws + neigh @ wn + b
    return jnp.maximum(h, 0.0)


def _tc_mid_body(agg2_ref, dsl_ref, x_ref, ws_ref, wn_ref, b_ref,
                 gamma_ref, beta_ref, out_ref):
    h = _neigh_h(agg2_ref[...], dsl_ref[...], x_ref[...],
                 ws_ref[...], wn_ref[...], b_ref[...])
    rid = lax.broadcasted_iota(jnp.int32, (_NP, 1), 0)
    valid = rid < _N
    h = jnp.where(valid, h, 0.0)
    mu = jnp.sum(h, axis=0, keepdims=True) * (1.0 / _N)
    cen = jnp.where(valid, h - mu, 0.0)
    var = jnp.sum(cen * cen, axis=0, keepdims=True) * (1.0 / _N)
    hn = cen * lax.rsqrt(var + 1e-5) * gamma_ref[...] + beta_ref[...]
    out_ref[...] = jnp.where(valid, hn, 0.0)


_tc_mid = pl.pallas_call(
    _tc_mid_body,
    out_shape=jax.ShapeDtypeStruct((_NP, 128), jnp.float32),
)


def _tc_head_body(agg2_ref, dsl_ref, x_ref, ws_ref, wn_ref, b_ref, gi_ref,
                  fc1w_ref, fc1b_ref, fc2w_ref, fc2b_ref, fc3w_ref, fc3b_ref,
                  out_ref):
    h = _neigh_h(agg2_ref[...], dsl_ref[...], x_ref[...],
                 ws_ref[...], wn_ref[...], b_ref[...])
    oh = (lax.broadcasted_iota(jnp.int32, (_G, _NP), 0)
          == gi_ref[...]).astype(jnp.float32)
    pooled = jnp.dot(oh, h, preferred_element_type=jnp.float32)
    h1 = jnp.maximum(jnp.dot(pooled, fc1w_ref[...],
                             preferred_element_type=jnp.float32)
                     + fc1b_ref[...], 0.0)
    h2 = jnp.maximum(jnp.dot(h1, fc2w_ref[...],
                             preferred_element_type=jnp.float32)
                     + fc2b_ref[...], 0.0)
    out_ref[...] = jnp.dot(h2, fc3w_ref[...],
                           preferred_element_type=jnp.float32) + fc3b_ref[...]


_tc_head = pl.pallas_call(
    _tc_head_body,
    out_shape=jax.ShapeDtypeStruct((_G, 10), jnp.float32),
)


def kernel(adjacency, input_feature, graph_indicator, labels, gamma2, beta2,
           gamma3, beta3, ws1, wn1, b1, ws2, wn2, b2, ws3, wn3, b3,
           fc1_w, fc1_b, fc2_w, fc2_b, fc3_w, fc3_b):
    srcr1, dstr1 = _edge_arrays(adjacency, 144)
    srcr2, dstr2 = _edge_arrays(adjacency, 128)
    x_pad = jnp.pad(input_feature, ((0, _NP - _N), (0, 0)))
    table1 = jnp.pad(
        jnp.concatenate([input_feature, jnp.ones((_N, 1), jnp.float32)],
                        axis=1),
        ((0, _NP - _N), (0, 15)))
    zeros144 = jnp.zeros((_NP, 144), jnp.float32)
    zeros128 = jnp.zeros((_NP, 128), jnp.float32)
    gi_pad = jnp.pad(graph_indicator, (0, _NP - _N),
                     constant_values=_G).reshape(1, _NP)
    b1r, b2r, b3r = (b.reshape(1, -1) for b in (b1, b2, b3))
    g2r, be2r = gamma2.reshape(1, -1), beta2.reshape(1, -1)
    g3r, be3r = gamma3.reshape(1, -1), beta3.reshape(1, -1)
    fc1br, fc2br, fc3br = (b.reshape(1, -1) for b in (fc1_b, fc2_b, fc3_b))

    p1 = _make_spmm(144)(table1, srcr1, dstr1, zeros144)
    agg1 = p1[:, :, :128]
    dsl = p1[:, :, 128:]
    h1 = _tc_mid(agg1, dsl, x_pad, ws1, wn1, b1r, g2r, be2r)
    p2 = _make_spmm(128)(h1, srcr2, dstr2, zeros128)
    h2 = _tc_mid(p2, dsl, h1, ws2, wn2, b2r, g3r, be3r)
    p3 = _make_spmm(128)(h2, srcr2, dstr2, zeros128)
    logits = _tc_head(p3, dsl, h2, ws3, wn3, b3r, gi_pad,
                      fc1_w, fc1br, fc2_w, fc2br, fc3_w, fc3br)
    return logits


# speedup vs baseline: 1.1557x; 1.1557x over previous
import functools

import jax
import jax.numpy as jnp
from jax import lax
from jax.experimental import pallas as pl
from jax.experimental.pallas import tpu as pltpu, tpu_sc as plsc

_N = 10000
_NP = 10016
_RSTRIPE = _NP // 16
_E = 320000
_NW = 32
_G = 64

_LAYOUT = {144: (64, 174, 142), 128: (80, 168, 84)}


def _edge_arrays(adjacency, width):
    lb, j0, j1 = _LAYOUT[width]
    ep = 16 * (j0 + j1) * lb
    pad = jnp.full((ep - _E,), _N, jnp.int32)
    srcr = jnp.concatenate([adjacency[0], pad]).reshape(16 * (j0 + j1), lb)
    dstr = jnp.concatenate([adjacency[1], pad]).reshape(16 * (j0 + j1), lb)
    return srcr, dstr


@functools.lru_cache(maxsize=None)
def _make_spmm(width):
    lb, j0, j1 = _LAYOUT[width]
    jmax = max(j0, j1)
    mesh = plsc.VectorSubcoreMesh(core_axis_name="c", subcore_axis_name="s",
                                  num_cores=2, num_subcores=16)

    @functools.partial(
        pl.kernel,
        out_type=jax.ShapeDtypeStruct((2, _NP, width), jnp.float32),
        mesh=mesh,
        scratch_types=[
            pltpu.VMEM_SHARED((_NP, width), jnp.float32),
            pltpu.VMEM((jmax, lb), jnp.int32),
            pltpu.VMEM((jmax, lb), jnp.int32),
            pltpu.VMEM((lb, width), jnp.float32),
            pltpu.VMEM((lb, width), jnp.float32),
            pltpu.SemaphoreType.DMA,
            pltpu.SemaphoreType.DMA,
        ],
        compiler_params=pltpu.CompilerParams(use_tc_tiling_on_sc=False),
    )
    def spmm(table, srcr, dstr, zeros, out, agg_sh, idxs, idxd,
             rows0, rows1, sem0, sem1):
        c = lax.axis_index("c")
        s = lax.axis_index("s")
        pltpu.sync_copy(zeros.at[pl.ds(s * _RSTRIPE, _RSTRIPE)],
                        agg_sh.at[pl.ds(s * _RSTRIPE, _RSTRIPE)])
        jc = jnp.where(c == 0, j0, j1)
        base = jnp.where(c == 0, s * j0, 16 * j0 + s * j1)
        pltpu.sync_copy(srcr.at[pl.ds(base, jmax)], idxs)
        pltpu.sync_copy(dstr.at[pl.ds(base, jmax)], idxd)
        plsc.subcore_barrier()

        pltpu.async_copy(table.at[idxs.at[0]], rows0, sem0)

        def step(p, carry):
            j = p * 2
            pltpu.async_copy(table.at[idxs.at[j + 1]], rows1, sem1)
            pltpu.make_async_copy(table.at[idxs.at[j]], rows0, sem0).wait()
            pltpu.sync_copy(rows0, agg_sh.at[idxd.at[j]], add=True)

            @pl.when(p < jc // 2 - 1)
            def _():
                pltpu.async_copy(table.at[idxs.at[j + 2]], rows0, sem0)

            pltpu.make_async_copy(table.at[idxs.at[j + 1]], rows1, sem1).wait()
            pltpu.sync_copy(rows1, agg_sh.at[idxd.at[j + 1]], add=True)
            return carry

        lax.fori_loop(0, jc // 2, step, 0)
        plsc.subcore_barrier()
        pltpu.sync_copy(agg_sh.at[pl.ds(s * _RSTRIPE, _RSTRIPE)],
                        out.at[c].at[pl.ds(s * _RSTRIPE, _RSTRIPE)])

    return spmm


def _neigh_h(agg2, dsl, x, ws, wn, b):
    p = agg2[0] + agg2[1]
    deg = jnp.sum(dsl[0] + dsl[1], axis=1, keepdims=True)
    neigh = p * (1.0 / jnp.maximum(deg, 1.0))
    h = x @ ws + neigh @ wn + b
    return jnp.maximum(h, 0.0)


def _tc_mid_body(agg2_ref, dsl_ref, x_ref, ws_ref, wn_ref, b_ref,
                 gamma_ref, beta_ref, out_ref):
    h = _neigh_h(agg2_ref[...], dsl_ref[...], x_ref[...],
                 ws_ref[...], wn_ref[...], b_ref[...])
    rid = lax.broadcasted_iota(jnp.int32, (_NP, 1), 0)
    valid = rid < _N
    h = jnp.where(valid, h, 0.0)
    mu = jnp.sum(h, axis=0, keepdims=True) * (1.0 / _N)
    cen = jnp.where(valid, h - mu, 0.0)
    var = jnp.sum(cen * cen, axis=0, keepdims=True) * (1.0 / _N)
    hn = cen * lax.rsqrt(var + 1e-5) * gamma_ref[...] + beta_ref[...]
    out_ref[...] = jnp.where(valid, hn, 0.0)


_tc_mid = pl.pallas_call(
    _tc_mid_body,
    out_shape=jax.ShapeDtypeStruct((_NP, 128), jnp.float32),
)


def _tc_head_body(agg2_ref, dsl_ref, x_ref, ws_ref, wn_ref, b_ref, gi_ref,
                  fc1w_ref, fc1b_ref, fc2w_ref, fc2b_ref, fc3w_ref, fc3b_ref,
                  out_ref):
    h = _neigh_h(agg2_ref[...], dsl_ref[...], x_ref[...],
                 ws_ref[...], wn_ref[...], b_ref[...])
    oh = (lax.broadcasted_iota(jnp.int32, (_G, _NP), 0)
          == gi_ref[...]).astype(jnp.float32)
    pooled = jnp.dot(oh, h, preferred_element_type=jnp.float32)
    h1 = jnp.maximum(jnp.dot(pooled, fc1w_ref[...],
                             preferred_element_type=jnp.float32)
                     + fc1b_ref[...], 0.0)
    h2 = jnp.maximum(jnp.dot(h1, fc2w_ref[...],
                             preferred_element_type=jnp.float32)
                     + fc2b_ref[...], 0.0)
    out_ref[...] = jnp.dot(h2, fc3w_ref[...],
                           preferred_element_type=jnp.float32) + fc3b_ref[...]


_tc_head = pl.pallas_call(
    _tc_head_body,
    out_shape=jax.ShapeDtypeStruct((_G, 10), jnp.float32),
)


def kernel(adjacency, input_feature, graph_indicator, labels, gamma2, beta2,
           gamma3, beta3, ws1, wn1, b1, ws2, wn2, b2, ws3, wn3, b3,
           fc1_w, fc1_b, fc2_w, fc2_b, fc3_w, fc3_b):
    srcr1, dstr1 = _edge_arrays(adjacency, 144)
    srcr2, dstr2 = _edge_arrays(adjacency, 128)
    x_pad = jnp.pad(input_feature, ((0, _NP - _N), (0, 0)))
    table1 = jnp.pad(
        jnp.concatenate([input_feature, jnp.ones((_N, 1), jnp.float32)],
                        axis=1),
        ((0, _NP - _N), (0, 15)))
    zeros144 = jnp.zeros((_NP, 144), jnp.float32)
    zeros128 = jnp.zeros((_NP, 128), jnp.float32)
    gi_pad = jnp.pad(graph_indicator, (0, _NP - _N),
                     constant_values=_G).reshape(1, _NP)
    b1r, b2r, b3r = (b.reshape(1, -1) for b in (b1, b2, b3))
    g2r, be2r = gamma2.reshape(1, -1), beta2.reshape(1, -1)
    g3r, be3r = gamma3.reshape(1, -1), beta3.reshape(1, -1)
    fc1br, fc2br, fc3br = (b.reshape(1, -1) for b in (fc1_b, fc2_b, fc3_b))

    p1 = _make_spmm(144)(table1, srcr1, dstr1, zeros144)
    agg1 = p1[:, :, :128]
    dsl = p1[:, :, 128:]
    h1 = _tc_mid(agg1, dsl, x_pad, ws1, wn1, b1r, g2r, be2r)
    p2 = _make_spmm(128)(h1, srcr2, dstr2, zeros128)
    h2 = _tc_mid(p2, dsl, h1, ws2, wn2, b2r, g3r, be3r)
    p3 = _make_spmm(128)(h2, srcr2, dstr2, zeros128)
    logits = _tc_head(p3, dsl, h2, ws3, wn3, b3r, gi_pad,
                      fc1_w, fc1br, fc2_w, fc2br, fc3_w, fc3br)
    return logits
